# baseline (speedup 1.0000x reference)
import functools

import jax
import jax.numpy as jnp
from jax import lax
from jax.experimental import pallas as pl
from jax.experimental.pallas import tpu as pltpu
from jax.experimental.pallas import tpu_sc as plsc

N = 10000
E = 320000
C = 128

NC = 2
NS = 16
NW = NC * NS
EPW = E // NW
CH = C // 2
EPT = E // NS
B = 128
NBF = EPT // B
TAIL = EPT - NBF * B
HB = NBF // 2
HT = HB // 3
ZR = 1000
LG = CH // 16

TC_BLK = 1000
TC_GRID = N // TC_BLK


def _proj_body(x_ref, w_ref, att2_ref, h_ref, asd_ref):
    h = jnp.dot(x_ref[...], w_ref[...], preferred_element_type=jnp.float32)
    h_ref[...] = h
    asd_ref[...] = jnp.dot(h, att2_ref[...], preferred_element_type=jnp.float32)


def _proj(x, w, att2):
    return pl.pallas_call(
        _proj_body,
        grid=(TC_GRID,),
        in_specs=[
            pl.BlockSpec((TC_BLK, C), lambda i: (i, 0)),
            pl.BlockSpec((C, C), lambda i: (0, 0)),
            pl.BlockSpec((C, 2), lambda i: (0, 0)),
        ],
        out_specs=[
            pl.BlockSpec((TC_BLK, C), lambda i: (i, 0)),
            pl.BlockSpec((TC_BLK, 2), lambda i: (i, 0)),
        ],
        out_shape=[
            jax.ShapeDtypeStruct((N, C), jnp.float32),
            jax.ShapeDtypeStruct((N, 2), jnp.float32),
        ],
    )(x, w, att2)


def _combine_temp(acc_ref, dent_ref, asd_ref, h_ref, bias_ref):
    a_s = asd_ref[:, 0:1]
    a_d = asd_ref[:, 1:2]
    es = a_s + a_d
    es = jnp.where(es >= 0, es, 0.2 * es)
    exs = jnp.exp(es)
    den = 0.5 * jnp.sum(dent_ref[...], axis=1, keepdims=True) + exs + 1e-16
    num = jnp.concatenate([acc_ref[0], acc_ref[1]], axis=1) + exs * h_ref[...]
    return num / den + bias_ref[...]


def _combine_mid_body(acc_ref, dent_ref, asd_ref, h_ref, bias_ref, w_ref,
                      att2_ref, h2_ref, asd2_ref):
    temp = _combine_temp(acc_ref, dent_ref, asd_ref, h_ref, bias_ref)
    y = jnp.where(temp >= 0, temp, 0.01 * temp) + temp
    h2 = jnp.dot(y, w_ref[...], preferred_element_type=jnp.float32)
    h2_ref[...] = h2
    asd2_ref[...] = jnp.dot(h2, att2_ref[...], preferred_element_type=jnp.float32)


def _combine_mid(acc, dent, asd, h, bias2, w, att2):
    return pl.pallas_call(
        _combine_mid_body,
        grid=(TC_GRID,),
        in_specs=[
            pl.BlockSpec((2, TC_BLK, CH), lambda i: (0, i, 0)),
            pl.BlockSpec((TC_BLK, NW), lambda i: (i, 0)),
            pl.BlockSpec((TC_BLK, 2), lambda i: (i, 0)),
            pl.BlockSpec((TC_BLK, C), lambda i: (i, 0)),
            pl.BlockSpec((1, C), lambda i: (0, 0)),
            pl.BlockSpec((C, C), lambda i: (0, 0)),
            pl.BlockSpec((C, 2), lambda i: (0, 0)),
        ],
        out_specs=[
            pl.BlockSpec((TC_BLK, C), lambda i: (i, 0)),
            pl.BlockSpec((TC_BLK, 2), lambda i: (i, 0)),
        ],
        out_shape=[
            jax.ShapeDtypeStruct((N, C), jnp.float32),
            jax.ShapeDtypeStruct((N, 2), jnp.float32),
        ],
    )(acc, dent, asd, h, bias2, w, att2)


def _combine_final_body(acc_ref, dent_ref, asd_ref, h_ref, bias_ref, out_ref):
    temp = _combine_temp(acc_ref, dent_ref, asd_ref, h_ref, bias_ref)
    out_ref[...] = jnp.where(temp >= 0, temp, 0.01 * temp)


def _combine_final(acc, dent, asd, h, bias2):
    return pl.pallas_call(
        _combine_final_body,
        grid=(TC_GRID,),
        in_specs=[
            pl.BlockSpec((2, TC_BLK, CH), lambda i: (0, i, 0)),
            pl.BlockSpec((TC_BLK, NW), lambda i: (i, 0)),
            pl.BlockSpec((TC_BLK, 2), lambda i: (i, 0)),
            pl.BlockSpec((TC_BLK, C), lambda i: (i, 0)),
            pl.BlockSpec((1, C), lambda i: (0, 0)),
        ],
        out_specs=pl.BlockSpec((TC_BLK, C), lambda i: (i, 0)),
        out_shape=jax.ShapeDtypeStruct((N, C), jnp.float32),
    )(acc, dent, asd, h, bias2)


_SC_MESH = plsc.VectorSubcoreMesh(core_axis_name="c", subcore_axis_name="s")


@functools.partial(
    pl.kernel,
    out_type=[
        jax.ShapeDtypeStruct((NC, N, CH), jnp.float32),
        jax.ShapeDtypeStruct((NW * N,), jnp.float32),
    ],
    mesh=_SC_MESH,
    compiler_params=pltpu.CompilerParams(needs_layout_passes=False,
                                         use_tc_tiling_on_sc=False),
    scratch_types=[
        pltpu.VMEM((HB, 2, B), jnp.int32),
        pltpu.VMEM((2, TAIL), jnp.int32),
        pltpu.VMEM((N,), jnp.float32),
        pltpu.VMEM((N,), jnp.float32),
        pltpu.VMEM((N,), jnp.float32),
        pltpu.VMEM((B, CH), jnp.float32),
        pltpu.VMEM((B, CH), jnp.float32),
        pltpu.VMEM((B, CH), jnp.float32),
        pltpu.VMEM_SHARED((N, CH), jnp.float32),
        pltpu.SemaphoreType.DMA,
        pltpu.SemaphoreType.DMA,
        pltpu.SemaphoreType.DMA,
        pltpu.SemaphoreType.DMA,
        pltpu.SemaphoreType.DMA,
        pltpu.SemaphoreType.DMA,
    ],
)
def _sc_edges(h_hbm, a_s_hbm, a_d_hbm, er_hbm, ert_hbm,
              zacc_hbm, acc_out, den_out,
              idx_v, idxt, as_v, ad_v, den_v, rows0, rows1, rows2,
              acc_sh, gsem0, gsem1, gsem2, ssem0, ssem1, ssem2):
    c = lax.axis_index("c")
    s = lax.axis_index("s")
    wid = c * NS + s

    pltpu.sync_copy(a_s_hbm, as_v)
    pltpu.sync_copy(a_d_hbm, ad_v)

    @pl.when(s < N // ZR)
    def _():
        pltpu.sync_copy(zacc_hbm.at[pl.ds(s * ZR, ZR)],
                        acc_sh.at[pl.ds(s * ZR, ZR)])

    zero16 = jnp.zeros((16,), jnp.float32)

    @pl.loop(0, N // 16)
    def _(i):
        den_v[pl.ds(i * 16, 16)] = zero16

    plsc.subcore_barrier()

    hc_hbm = h_hbm.at[c]
    rows = (rows0, rows1, rows2)
    gsem = (gsem0, gsem1, gsem2)
    ssem = (ssem0, ssem1, ssem2)

    def start_gather(lb, i):
        pltpu.async_copy(hc_hbm.at[idx_v.at[lb, 0]], rows[i], gsem[i])

    def wait_gather(lb, i):
        pltpu.make_async_copy(hc_hbm.at[idx_v.at[lb, 0]], rows[i],
                              gsem[i]).wait()

    def start_scatter(lb, i):
        pltpu.async_copy(rows[i], acc_sh.at[idx_v.at[lb, 1]], ssem[i],
                         add=True)

    def wait_scatter(lb, i):
        pltpu.make_async_copy(rows[i], acc_sh.at[idx_v.at[lb, 1]],
                              ssem[i]).wait()

    def compute(idx_b, rows_v, ngroups=B // 16):
        @pl.loop(0, ngroups)
        def _(g):
            off = g * 16
            s16 = idx_b[0, pl.ds(off, 16)]
            d16 = idx_b[1, pl.ds(off, 16)]
            e = plsc.load_gather(as_v, [s16]) + plsc.load_gather(ad_v, [d16])
            e = jnp.where(e >= 0, e, 0.2 * e)
            ex = jnp.exp(e)
            plsc.addupdate_scatter(den_v, [d16], ex)
            for j in range(16):
                w = ex[j]
                r = off + j
                for lg in range(LG):
                    sl = pl.ds(lg * 16, 16)
                    rows_v[r, sl] = rows_v[r, sl] * w

    def run_batch(lb, i):
        wait_gather(lb, i)
        compute(idx_v.at[lb], rows[i])
        start_scatter(lb, i)

    def half(hh):
        pltpu.sync_copy(er_hbm.at[s, hh], idx_v)
        start_gather(0, 0)
        start_gather(1, 1)
        run_batch(0, 0)
        start_gather(2, 2)

        @pl.loop(0, HT - 1)
        def _(j):
            lb = 3 * j
            run_batch(lb + 1, 1)
            wait_scatter(lb, 0)
            start_gather(lb + 3, 0)
            run_batch(lb + 2, 2)
            wait_scatter(lb + 1, 1)
            start_gather(lb + 4, 1)
            run_batch(lb + 3, 0)
            wait_scatter(lb + 2, 2)
            start_gather(lb + 5, 2)

        run_batch(HB - 2, 1)
        wait_scatter(HB - 3, 0)
        run_batch(HB - 1, 2)
        wait_scatter(HB - 2, 1)
        wait_scatter(HB - 1, 2)

    half(0)
    half(1)

    pltpu.sync_copy(ert_hbm.at[s], idxt)
    rowst = rows0.at[pl.ds(0, TAIL)]
    pltpu.async_copy(hc_hbm.at[idxt.at[0]], rowst, gsem0).wait()
    compute(idxt, rows0, ngroups=TAIL // 16)
    pltpu.async_copy(rowst, acc_sh.at[idxt.at[1]], ssem0, add=True).wait()

    pltpu.sync_copy(den_v, den_out.at[pl.ds(wid * N, N)])

    plsc.subcore_barrier()

    @pl.when(s < N // ZR)
    def _():
        pltpu.sync_copy(acc_sh.at[pl.ds(s * ZR, ZR)],
                        acc_out.at[c, pl.ds(s * ZR, ZR)])


def kernel(x, edges, W, att_src, att_dst, bias):
    att2 = jnp.stack([att_src, att_dst], axis=1)
    bias2 = bias.reshape(1, C)
    src_c = edges[0].reshape(NS, EPT)
    dst_c = edges[1].reshape(NS, EPT)
    er = jnp.stack([src_c[:, :NBF * B].reshape(NS, NBF, B),
                    dst_c[:, :NBF * B].reshape(NS, NBF, B)],
                   axis=2).reshape(NS, 2, HB, 2, B)
    ert = jnp.stack([src_c[:, NBF * B:], dst_c[:, NBF * B:]], axis=1)
    zacc = jnp.zeros((N, CH), jnp.float32)

    h1, asd1 = _proj(x, W, att2)
    h1h = jnp.stack([h1[:, :CH], h1[:, CH:]])
    acc1, den1 = _sc_edges(h1h, asd1[:, 0], asd1[:, 1], er, ert, zacc)
    h2, asd2 = _combine_mid(acc1, den1.reshape(NW, N).T, asd1, h1, bias2, W,
                            att2)
    h2h = jnp.stack([h2[:, :CH], h2[:, CH:]])
    acc2, den2 = _sc_edges(h2h, asd2[:, 0], asd2[:, 1], er, ert, zacc)
    return _combine_final(acc2, den2.reshape(NW, N).T, asd2, h2, bias2)


# speedup vs baseline: 1.5826x; 1.5826x over previous
import functools

import jax
import jax.numpy as jnp
from jax import lax
from jax.experimental import pallas as pl
from jax.experimental.pallas import tpu as pltpu
from jax.experimental.pallas import tpu_sc as plsc

N = 10000
E = 320000
C = 128

NC = 2
NS = 16
NW = NC * NS
EPW = E // NW
CH = C // 2
EPT = E // NS
B = 128
NBF = EPT // B
TAIL = EPT - NBF * B
HB = NBF // 2
HT = HB // 3
ZR = 1000
LG = CH // 16

TC_BLK = 1000
TC_GRID = N // TC_BLK


def _proj_body(x_ref, w_ref, att2_ref, h_ref, asd_ref):
    h = jnp.dot(x_ref[...], w_ref[...], preferred_element_type=jnp.float32)
    h_ref[...] = h
    asd_ref[...] = jnp.dot(h, att2_ref[...], preferred_element_type=jnp.float32)


def _proj(x, w, att2):
    return pl.pallas_call(
        _proj_body,
        grid=(TC_GRID,),
        in_specs=[
            pl.BlockSpec((TC_BLK, C), lambda i: (i, 0)),
            pl.BlockSpec((C, C), lambda i: (0, 0)),
            pl.BlockSpec((C, 2), lambda i: (0, 0)),
        ],
        out_specs=[
            pl.BlockSpec((TC_BLK, C), lambda i: (i, 0)),
            pl.BlockSpec((TC_BLK, 2), lambda i: (i, 0)),
        ],
        out_shape=[
            jax.ShapeDtypeStruct((N, C), jnp.float32),
            jax.ShapeDtypeStruct((N, 2), jnp.float32),
        ],
    )(x, w, att2)


def _combine_temp(acc_ref, dent_ref, asd_ref, h_ref, bias_ref):
    a_s = asd_ref[:, 0:1]
    a_d = asd_ref[:, 1:2]
    es = a_s + a_d
    es = jnp.where(es >= 0, es, 0.2 * es)
    exs = jnp.exp(es)
    den = 0.5 * jnp.sum(dent_ref[...], axis=1, keepdims=True) + exs + 1e-16
    num = jnp.concatenate([acc_ref[0], acc_ref[1]], axis=1) + exs * h_ref[...]
    return num / den + bias_ref[...]


def _combine_mid_body(acc_ref, dent_ref, asd_ref, h_ref, bias_ref, w_ref,
                      att2_ref, h2_ref, asd2_ref):
    temp = _combine_temp(acc_ref, dent_ref, asd_ref, h_ref, bias_ref)
    y = jnp.where(temp >= 0, temp, 0.01 * temp) + temp
    h2 = jnp.dot(y, w_ref[...], preferred_element_type=jnp.float32)
    h2_ref[...] = h2
    asd2_ref[...] = jnp.dot(h2, att2_ref[...], preferred_element_type=jnp.float32)


def _combine_mid(acc, dent, asd, h, bias2, w, att2):
    return pl.pallas_call(
        _combine_mid_body,
        grid=(TC_GRID,),
        in_specs=[
            pl.BlockSpec((2, TC_BLK, CH), lambda i: (0, i, 0)),
            pl.BlockSpec((TC_BLK, NW), lambda i: (i, 0)),
            pl.BlockSpec((TC_BLK, 2), lambda i: (i, 0)),
            pl.BlockSpec((TC_BLK, C), lambda i: (i, 0)),
            pl.BlockSpec((1, C), lambda i: (0, 0)),
            pl.BlockSpec((C, C), lambda i: (0, 0)),
            pl.BlockSpec((C, 2), lambda i: (0, 0)),
        ],
        out_specs=[
            pl.BlockSpec((TC_BLK, C), lambda i: (i, 0)),
            pl.BlockSpec((TC_BLK, 2), lambda i: (i, 0)),
        ],
        out_shape=[
            jax.ShapeDtypeStruct((N, C), jnp.float32),
            jax.ShapeDtypeStruct((N, 2), jnp.float32),
        ],
    )(acc, dent, asd, h, bias2, w, att2)


def _combine_final_body(acc_ref, dent_ref, asd_ref, h_ref, bias_ref, out_ref):
    temp = _combine_temp(acc_ref, dent_ref, asd_ref, h_ref, bias_ref)
    out_ref[...] = jnp.where(temp >= 0, temp, 0.01 * temp)


def _combine_final(acc, dent, asd, h, bias2):
    return pl.pallas_call(
        _combine_final_body,
        grid=(TC_GRID,),
        in_specs=[
            pl.BlockSpec((2, TC_BLK, CH), lambda i: (0, i, 0)),
            pl.BlockSpec((TC_BLK, NW), lambda i: (i, 0)),
            pl.BlockSpec((TC_BLK, 2), lambda i: (i, 0)),
            pl.BlockSpec((TC_BLK, C), lambda i: (i, 0)),
            pl.BlockSpec((1, C), lambda i: (0, 0)),
        ],
        out_specs=pl.BlockSpec((TC_BLK, C), lambda i: (i, 0)),
        out_shape=jax.ShapeDtypeStruct((N, C), jnp.float32),
    )(acc, dent, asd, h, bias2)


_SC_MESH = plsc.VectorSubcoreMesh(core_axis_name="c", subcore_axis_name="s")


@functools.partial(
    pl.kernel,
    out_type=[
        jax.ShapeDtypeStruct((NC, N, CH), jnp.float32),
        jax.ShapeDtypeStruct((NW * N,), jnp.float32),
    ],
    mesh=_SC_MESH,
    compiler_params=pltpu.CompilerParams(needs_layout_passes=False,
                                         use_tc_tiling_on_sc=False),
    scratch_types=[
        pltpu.VMEM((HB, 2, B), jnp.int32),
        pltpu.VMEM((2, TAIL), jnp.int32),
        pltpu.VMEM((N,), jnp.float32),
        pltpu.VMEM((N,), jnp.float32),
        pltpu.VMEM((N,), jnp.float32),
        pltpu.VMEM((B, CH), jnp.float32),
        pltpu.VMEM((B, CH), jnp.float32),
        pltpu.VMEM((B, CH), jnp.float32),
        pltpu.VMEM_SHARED((N, CH), jnp.float32),
        pltpu.SemaphoreType.DMA,
        pltpu.SemaphoreType.DMA,
        pltpu.SemaphoreType.DMA,
        pltpu.SemaphoreType.DMA,
        pltpu.SemaphoreType.DMA,
        pltpu.SemaphoreType.DMA,
    ],
)
def _sc_edges(h_hbm, a_s_hbm, a_d_hbm, er_hbm, ert_hbm,
              zacc_hbm, acc_out, den_out,
              idx_v, idxt, as_v, ad_v, den_v, rows0, rows1, rows2,
              acc_sh, gsem0, gsem1, gsem2, ssem0, ssem1, ssem2):
    c = lax.axis_index("c")
    s = lax.axis_index("s")
    wid = c * NS + s

    pltpu.sync_copy(a_s_hbm, as_v)
    pltpu.sync_copy(a_d_hbm, ad_v)

    @pl.when(s < N // ZR)
    def _():
        pltpu.sync_copy(zacc_hbm.at[pl.ds(s * ZR, ZR)],
                        acc_sh.at[pl.ds(s * ZR, ZR)])

    zero16 = jnp.zeros((16,), jnp.float32)

    @pl.loop(0, N // 16)
    def _(i):
        den_v[pl.ds(i * 16, 16)] = zero16

    plsc.subcore_barrier()

    hc_hbm = h_hbm.at[c]
    rows = (rows0, rows1, rows2)
    gsem = (gsem0, gsem1, gsem2)
    ssem = (ssem0, ssem1, ssem2)

    def start_gather(lb, i):
        pltpu.async_copy(hc_hbm.at[idx_v.at[lb, 0]], rows[i], gsem[i])

    def wait_gather(lb, i):
        pltpu.make_async_copy(hc_hbm.at[idx_v.at[lb, 0]], rows[i],
                              gsem[i]).wait()

    def start_scatter(lb, i):
        pltpu.async_copy(rows[i], acc_sh.at[idx_v.at[lb, 1]], ssem[i],
                         add=True)

    def wait_scatter(lb, i):
        pltpu.make_async_copy(rows[i], acc_sh.at[idx_v.at[lb, 1]],
                              ssem[i]).wait()

    def compute(idx_b, rows_v, ngroups=B // 16):
        for g in range(ngroups):
            off = g * 16
            s16 = idx_b[0, pl.ds(off, 16)]
            d16 = idx_b[1, pl.ds(off, 16)]
            e = plsc.load_gather(as_v, [s16]) + plsc.load_gather(ad_v, [d16])
            e = jnp.where(e >= 0, e, 0.2 * e)
            ex = jnp.exp(e)
            plsc.addupdate_scatter(den_v, [d16], ex)
            for j in range(16):
                w = ex[j]
                r = off + j
                for lg in range(LG):
                    sl = pl.ds(lg * 16, 16)
                    rows_v[r, sl] = rows_v[r, sl] * w

    def run_batch(lb, i):
        wait_gather(lb, i)
        compute(idx_v.at[lb], rows[i])
        start_scatter(lb, i)

    @pl.loop(0, 2)
    def half(hh):
        pltpu.sync_copy(er_hbm.at[s, hh], idx_v)
        start_gather(0, 0)
        start_gather(1, 1)
        run_batch(0, 0)
        start_gather(2, 2)

        @pl.loop(0, HT - 1)
        def _(j):
            lb = 3 * j
            run_batch(lb + 1, 1)
            wait_scatter(lb, 0)
            start_gather(lb + 3, 0)
            run_batch(lb + 2, 2)
            wait_scatter(lb + 1, 1)
            start_gather(lb + 4, 1)
            run_batch(lb + 3, 0)
            wait_scatter(lb + 2, 2)
            start_gather(lb + 5, 2)

        run_batch(HB - 2, 1)
        wait_scatter(HB - 3, 0)
        run_batch(HB - 1, 2)
        wait_scatter(HB - 2, 1)
        wait_scatter(HB - 1, 2)

    pltpu.sync_copy(ert_hbm.at[s], idxt)
    rowst = rows0.at[pl.ds(0, TAIL)]
    pltpu.async_copy(hc_hbm.at[idxt.at[0]], rowst, gsem0).wait()
    compute(idxt, rows0, ngroups=TAIL // 16)
    pltpu.async_copy(rowst, acc_sh.at[idxt.at[1]], ssem0, add=True).wait()

    pltpu.sync_copy(den_v, den_out.at[pl.ds(wid * N, N)])

    plsc.subcore_barrier()

    @pl.when(s < N // ZR)
    def _():
        pltpu.sync_copy(acc_sh.at[pl.ds(s * ZR, ZR)],
                        acc_out.at[c, pl.ds(s * ZR, ZR)])


def kernel(x, edges, W, att_src, att_dst, bias):
    att2 = jnp.stack([att_src, att_dst], axis=1)
    bias2 = bias.reshape(1, C)
    src_c = edges[0].reshape(NS, EPT)
    dst_c = edges[1].reshape(NS, EPT)
    er = jnp.stack([src_c[:, :NBF * B].reshape(NS, NBF, B),
                    dst_c[:, :NBF * B].reshape(NS, NBF, B)],
                   axis=2).reshape(NS, 2, HB, 2, B)
    ert = jnp.stack([src_c[:, NBF * B:], dst_c[:, NBF * B:]], axis=1)
    zacc = jnp.zeros((N, CH), jnp.float32)

    h1, asd1 = _proj(x, W, att2)
    h1h = jnp.stack([h1[:, :CH], h1[:, CH:]])
    acc1, den1 = _sc_edges(h1h, asd1[:, 0], asd1[:, 1], er, ert, zacc)
    h2, asd2 = _combine_mid(acc1, den1.reshape(NW, N).T, asd1, h1, bias2, W,
                            att2)
    h2h = jnp.stack([h2[:, :CH], h2[:, CH:]])
    acc2, den2 = _sc_edges(h2h, asd2[:, 0], asd2[:, 1], er, ert, zacc)
    return _combine_final(acc2, den2.reshape(NW, N).T, asd2, h2, bias2)


# speedup vs baseline: 1.9177x; 1.2117x over previous
import functools

import jax
import jax.numpy as jnp
from jax import lax
from jax.experimental import pallas as pl
from jax.experimental.pallas import tpu as pltpu
from jax.experimental.pallas import tpu_sc as plsc

N = 10000
E = 320000
C = 128

NC = 2
NS = 16
NW = NC * NS
EPW = E // NW
CH = C // 2
EPT = E // NS
B = 64
NBF = EPT // B
TAIL = EPT - NBF * B
STEADY = NBF - 6
ZR = 1000
LG = CH // 16

TC_BLK = 1000
TC_GRID = N // TC_BLK


def _proj_body(x_ref, w_ref, att2_ref, h_ref, asd_ref):
    h = jnp.dot(x_ref[...], w_ref[...], preferred_element_type=jnp.float32)
    h_ref[...] = h
    asd_ref[...] = jnp.dot(h, att2_ref[...], preferred_element_type=jnp.float32)


def _proj(x, w, att2):
    return pl.pallas_call(
        _proj_body,
        grid=(TC_GRID,),
        in_specs=[
            pl.BlockSpec((TC_BLK, C), lambda i: (i, 0)),
            pl.BlockSpec((C, C), lambda i: (0, 0)),
            pl.BlockSpec((C, 2), lambda i: (0, 0)),
        ],
        out_specs=[
            pl.BlockSpec((TC_BLK, C), lambda i: (i, 0)),
            pl.BlockSpec((TC_BLK, 2), lambda i: (i, 0)),
        ],
        out_shape=[
            jax.ShapeDtypeStruct((N, C), jnp.float32),
            jax.ShapeDtypeStruct((N, 2), jnp.float32),
        ],
    )(x, w, att2)


def _combine_temp(acc_ref, dent_ref, asd_ref, h_ref, bias_ref):
    a_s = asd_ref[:, 0:1]
    a_d = asd_ref[:, 1:2]
    es = a_s + a_d
    es = jnp.where(es >= 0, es, 0.2 * es)
    exs = jnp.exp(es)
    den = 0.5 * jnp.sum(dent_ref[...], axis=1, keepdims=True) + exs + 1e-16
    num = jnp.concatenate([acc_ref[0], acc_ref[1]], axis=1) + exs * h_ref[...]
    return num / den + bias_ref[...]


def _combine_mid_body(acc_ref, dent_ref, asd_ref, h_ref, bias_ref, w_ref,
                      att2_ref, h2_ref, asd2_ref):
    temp = _combine_temp(acc_ref, dent_ref, asd_ref, h_ref, bias_ref)
    y = jnp.where(temp >= 0, temp, 0.01 * temp) + temp
    h2 = jnp.dot(y, w_ref[...], preferred_element_type=jnp.float32)
    h2_ref[...] = h2
    asd2_ref[...] = jnp.dot(h2, att2_ref[...], preferred_element_type=jnp.float32)


def _combine_mid(acc, dent, asd, h, bias2, w, att2):
    return pl.pallas_call(
        _combine_mid_body,
        grid=(TC_GRID,),
        in_specs=[
            pl.BlockSpec((2, TC_BLK, CH), lambda i: (0, i, 0)),
            pl.BlockSpec((TC_BLK, NW), lambda i: (i, 0)),
            pl.BlockSpec((TC_BLK, 2), lambda i: (i, 0)),
            pl.BlockSpec((TC_BLK, C), lambda i: (i, 0)),
            pl.BlockSpec((1, C), lambda i: (0, 0)),
            pl.BlockSpec((C, C), lambda i: (0, 0)),
            pl.BlockSpec((C, 2), lambda i: (0, 0)),
        ],
        out_specs=[
            pl.BlockSpec((TC_BLK, C), lambda i: (i, 0)),
            pl.BlockSpec((TC_BLK, 2), lambda i: (i, 0)),
        ],
        out_shape=[
            jax.ShapeDtypeStruct((N, C), jnp.float32),
            jax.ShapeDtypeStruct((N, 2), jnp.float32),
        ],
    )(acc, dent, asd, h, bias2, w, att2)


def _combine_final_body(acc_ref, dent_ref, asd_ref, h_ref, bias_ref, out_ref):
    temp = _combine_temp(acc_ref, dent_ref, asd_ref, h_ref, bias_ref)
    out_ref[...] = jnp.where(temp >= 0, temp, 0.01 * temp)


def _combine_final(acc, dent, asd, h, bias2):
    return pl.pallas_call(
        _combine_final_body,
        grid=(TC_GRID,),
        in_specs=[
            pl.BlockSpec((2, TC_BLK, CH), lambda i: (0, i, 0)),
            pl.BlockSpec((TC_BLK, NW), lambda i: (i, 0)),
            pl.BlockSpec((TC_BLK, 2), lambda i: (i, 0)),
            pl.BlockSpec((TC_BLK, C), lambda i: (i, 0)),
            pl.BlockSpec((1, C), lambda i: (0, 0)),
        ],
        out_specs=pl.BlockSpec((TC_BLK, C), lambda i: (i, 0)),
        out_shape=jax.ShapeDtypeStruct((N, C), jnp.float32),
    )(acc, dent, asd, h, bias2)


_SC_MESH = plsc.VectorSubcoreMesh(core_axis_name="c", subcore_axis_name="s")


@functools.partial(
    pl.kernel,
    out_type=[
        jax.ShapeDtypeStruct((NC, N, CH), jnp.float32),
        jax.ShapeDtypeStruct((NW * N,), jnp.float32),
    ],
    mesh=_SC_MESH,
    compiler_params=pltpu.CompilerParams(needs_layout_passes=False,
                                         use_tc_tiling_on_sc=False),
    scratch_types=[
        pltpu.VMEM((B,), jnp.int32),
        pltpu.VMEM((B,), jnp.int32),
        pltpu.VMEM((B,), jnp.int32),
        pltpu.VMEM((B,), jnp.int32),
        pltpu.VMEM((B,), jnp.int32),
        pltpu.VMEM((B,), jnp.int32),
        pltpu.VMEM((2, TAIL), jnp.int32),
        pltpu.VMEM((N,), jnp.float32),
        pltpu.VMEM((N,), jnp.float32),
        pltpu.VMEM((N,), jnp.float32),
        pltpu.VMEM((B, CH), jnp.float32),
        pltpu.VMEM((B, CH), jnp.float32),
        pltpu.VMEM((B, CH), jnp.float32),
        pltpu.VMEM_SHARED((N, CH), jnp.float32),
        pltpu.VMEM_SHARED((N, CH), jnp.float32),
        pltpu.SemaphoreType.DMA,
        pltpu.SemaphoreType.DMA,
        pltpu.SemaphoreType.DMA,
        pltpu.SemaphoreType.DMA,
        pltpu.SemaphoreType.DMA,
        pltpu.SemaphoreType.DMA,
        pltpu.SemaphoreType.DMA,
        pltpu.SemaphoreType.DMA,
        pltpu.SemaphoreType.DMA,
        pltpu.SemaphoreType.DMA,
        pltpu.SemaphoreType.DMA,
        pltpu.SemaphoreType.DMA,
    ],
)
def _sc_edges(h_hbm, a_s_hbm, a_d_hbm, src_hbm, dst_hbm, ert_hbm,
              zacc_hbm, acc_out, den_out,
              srcv0, srcv1, srcv2, dstv0, dstv1, dstv2, idxt,
              as_v, ad_v, den_v, rows0, rows1, rows2,
              acc_sh, h_sh,
              gsem0, gsem1, gsem2, ssem0, ssem1, ssem2,
              isg0, isg1, isg2, isd0, isd1, isd2):
    c = lax.axis_index("c")
    s = lax.axis_index("s")
    wid = c * NS + s
    ebase = s * EPT

    pltpu.sync_copy(a_s_hbm, as_v)
    pltpu.sync_copy(a_d_hbm, ad_v)

    @pl.when(s < N // ZR)
    def _():
        pltpu.sync_copy(zacc_hbm.at[pl.ds(s * ZR, ZR)],
                        acc_sh.at[pl.ds(s * ZR, ZR)])
        pltpu.sync_copy(h_hbm.at[c, pl.ds(s * ZR, ZR)],
                        h_sh.at[pl.ds(s * ZR, ZR)])

    zero16 = jnp.zeros((16,), jnp.float32)

    @pl.loop(0, N // 16)
    def _(i):
        den_v[pl.ds(i * 16, 16)] = zero16

    plsc.subcore_barrier()

    rows = (rows0, rows1, rows2)
    srcv = (srcv0, srcv1, srcv2)
    dstv = (dstv0, dstv1, dstv2)
    gsem = (gsem0, gsem1, gsem2)
    ssem = (ssem0, ssem1, ssem2)
    isg = (isg0, isg1, isg2)
    isd = (isd0, isd1, isd2)

    def stage_src(b, i):
        pltpu.async_copy(src_hbm.at[pl.ds(ebase + b * B, B)], srcv[i], isg[i])

    def stage_dst(b, i):
        pltpu.async_copy(dst_hbm.at[pl.ds(ebase + b * B, B)], dstv[i], isd[i])

    def wait_src(i):
        pltpu.make_async_copy(src_hbm.at[pl.ds(0, B)], srcv[i], isg[i]).wait()

    def wait_dst(i):
        pltpu.make_async_copy(dst_hbm.at[pl.ds(0, B)], dstv[i], isd[i]).wait()

    def start_gather(i):
        pltpu.async_copy(h_sh.at[srcv[i]], rows[i], gsem[i])

    def wait_gather(i):
        pltpu.make_async_copy(h_sh.at[srcv[i]], rows[i], gsem[i]).wait()

    def start_scatter(i):
        pltpu.async_copy(rows[i], acc_sh.at[dstv[i]], ssem[i], add=True)

    def wait_scatter(i):
        pltpu.make_async_copy(rows[i], acc_sh.at[dstv[i]], ssem[i]).wait()

    def compute(src_b, dst_b, rows_v, ngroups=B // 16):
        for g in range(ngroups):
            off = g * 16
            s16 = src_b[pl.ds(off, 16)]
            d16 = dst_b[pl.ds(off, 16)]
            e = plsc.load_gather(as_v, [s16]) + plsc.load_gather(ad_v, [d16])
            e = jnp.where(e >= 0, e, 0.2 * e)
            ex = jnp.exp(e)
            plsc.addupdate_scatter(den_v, [d16], ex)
            for j in range(16):
                w = ex[j]
                r = off + j
                for lg in range(LG):
                    sl = pl.ds(lg * 16, 16)
                    rows_v[r, sl] = rows_v[r, sl] * w

    def head(i):
        wait_gather(i)
        wait_dst(i)
        compute(srcv[i], dstv[i], rows[i])
        start_scatter(i)

    def run(b, i):
        ip = (i + 2) % 3
        head(i)
        stage_src(b + 3, i)
        wait_scatter(ip)
        stage_dst(b + 2, ip)
        wait_src(ip)
        start_gather(ip)

    for i in range(3):
        stage_src(i, i)
        stage_dst(i, i)
    for i in range(3):
        wait_src(i)
        start_gather(i)

    head(0)
    stage_src(3, 0)

    @pl.loop(0, STEADY // 3)
    def _(t):
        b = 3 * t
        run(b + 1, 1)
        run(b + 2, 2)
        run(b + 3, 0)

    run(NBF - 5, 1)
    run(NBF - 4, 2)
    head(0)
    wait_scatter(2)
    stage_dst(NBF - 1, 2)
    wait_src(2)
    start_gather(2)
    head(1)
    wait_scatter(0)
    head(2)
    wait_scatter(1)
    wait_scatter(2)

    pltpu.sync_copy(ert_hbm.at[s], idxt)
    rowst = rows0.at[pl.ds(0, TAIL)]
    pltpu.async_copy(h_sh.at[idxt.at[0]], rowst, gsem0).wait()
    compute(idxt.at[0], idxt.at[1], rows0, ngroups=TAIL // 16)
    pltpu.async_copy(rowst, acc_sh.at[idxt.at[1]], ssem0, add=True).wait()

    pltpu.sync_copy(den_v, den_out.at[pl.ds(wid * N, N)])

    plsc.subcore_barrier()

    @pl.when(s < N // ZR)
    def _():
        pltpu.sync_copy(acc_sh.at[pl.ds(s * ZR, ZR)],
                        acc_out.at[c, pl.ds(s * ZR, ZR)])


def kernel(x, edges, W, att_src, att_dst, bias):
    att2 = jnp.stack([att_src, att_dst], axis=1)
    bias2 = bias.reshape(1, C)
    src_c = edges[0].reshape(NS, EPT)
    dst_c = edges[1].reshape(NS, EPT)
    ert = jnp.stack([src_c[:, NBF * B:], dst_c[:, NBF * B:]], axis=1)
    src_f = edges[0]
    dst_f = edges[1]
    zacc = jnp.zeros((N, CH), jnp.float32)

    h1, asd1 = _proj(x, W, att2)
    h1h = jnp.stack([h1[:, :CH], h1[:, CH:]])
    acc1, den1 = _sc_edges(h1h, asd1[:, 0], asd1[:, 1], src_f, dst_f, ert,
                           zacc)
    h2, asd2 = _combine_mid(acc1, den1.reshape(NW, N).T, asd1, h1, bias2, W,
                            att2)
    h2h = jnp.stack([h2[:, :CH], h2[:, CH:]])
    acc2, den2 = _sc_edges(h2h, asd2[:, 0], asd2[:, 1], src_f, dst_f, ert,
                           zacc)
    return _combine_final(acc2, den2.reshape(NW, N).T, asd2, h2, bias2)


# speedup vs baseline: 1.9445x; 1.0140x over previous
import functools

import jax
import jax.numpy as jnp
from jax import lax
from jax.experimental import pallas as pl
from jax.experimental.pallas import tpu as pltpu
from jax.experimental.pallas import tpu_sc as plsc

N = 10000
E = 320000
C = 128

NC = 2
NS = 16
NW = NC * NS
EPW = E // NW
CH = C // 2
EPT = E // NS
B = 64
NBF = EPT // B
TAIL = EPT - NBF * B
STEADY = NBF - 6
ZR = 1000
LG = CH // 16

TC_BLK = 1000
TC_GRID = N // TC_BLK


def _proj_body(x_ref, w_ref, att2_ref, h_ref, asd_ref):
    h = jnp.dot(x_ref[...], w_ref[...], preferred_element_type=jnp.float32)
    h_ref[0] = h[:, :CH]
    h_ref[1] = h[:, CH:]
    asd_ref[...] = jnp.dot(h, att2_ref[...], preferred_element_type=jnp.float32)


def _proj(x, w, att2):
    return pl.pallas_call(
        _proj_body,
        grid=(TC_GRID,),
        in_specs=[
            pl.BlockSpec((TC_BLK, C), lambda i: (i, 0)),
            pl.BlockSpec((C, C), lambda i: (0, 0)),
            pl.BlockSpec((C, 2), lambda i: (0, 0)),
        ],
        out_specs=[
            pl.BlockSpec((2, TC_BLK, CH), lambda i: (0, i, 0)),
            pl.BlockSpec((TC_BLK, 2), lambda i: (i, 0)),
        ],
        out_shape=[
            jax.ShapeDtypeStruct((2, N, CH), jnp.float32),
            jax.ShapeDtypeStruct((N, 2), jnp.float32),
        ],
    )(x, w, att2)


def _combine_temp(acc_ref, dent_ref, asd_ref, h_ref, bias_ref):
    a_s = asd_ref[:, 0:1]
    a_d = asd_ref[:, 1:2]
    es = a_s + a_d
    es = jnp.where(es >= 0, es, 0.2 * es)
    exs = jnp.exp(es)
    den = 0.5 * jnp.sum(dent_ref[...], axis=1, keepdims=True) + exs + 1e-16
    h = jnp.concatenate([h_ref[0], h_ref[1]], axis=1)
    num = jnp.concatenate([acc_ref[0], acc_ref[1]], axis=1) + exs * h
    return num / den + bias_ref[...]


def _combine_mid_body(acc_ref, dent_ref, asd_ref, h_ref, bias_ref, w_ref,
                      att2_ref, h2_ref, asd2_ref):
    temp = _combine_temp(acc_ref, dent_ref, asd_ref, h_ref, bias_ref)
    y = jnp.where(temp >= 0, temp, 0.01 * temp) + temp
    h2 = jnp.dot(y, w_ref[...], preferred_element_type=jnp.float32)
    h2_ref[0] = h2[:, :CH]
    h2_ref[1] = h2[:, CH:]
    asd2_ref[...] = jnp.dot(h2, att2_ref[...], preferred_element_type=jnp.float32)


def _combine_mid(acc, dent, asd, h, bias2, w, att2):
    return pl.pallas_call(
        _combine_mid_body,
        grid=(TC_GRID,),
        in_specs=[
            pl.BlockSpec((2, TC_BLK, CH), lambda i: (0, i, 0)),
            pl.BlockSpec((TC_BLK, NW), lambda i: (i, 0)),
            pl.BlockSpec((TC_BLK, 2), lambda i: (i, 0)),
            pl.BlockSpec((2, TC_BLK, CH), lambda i: (0, i, 0)),
            pl.BlockSpec((1, C), lambda i: (0, 0)),
            pl.BlockSpec((C, C), lambda i: (0, 0)),
            pl.BlockSpec((C, 2), lambda i: (0, 0)),
        ],
        out_specs=[
            pl.BlockSpec((2, TC_BLK, CH), lambda i: (0, i, 0)),
            pl.BlockSpec((TC_BLK, 2), lambda i: (i, 0)),
        ],
        out_shape=[
            jax.ShapeDtypeStruct((2, N, CH), jnp.float32),
            jax.ShapeDtypeStruct((N, 2), jnp.float32),
        ],
    )(acc, dent, asd, h, bias2, w, att2)


def _combine_final_body(acc_ref, dent_ref, asd_ref, h_ref, bias_ref, out_ref):
    temp = _combine_temp(acc_ref, dent_ref, asd_ref, h_ref, bias_ref)
    out_ref[...] = jnp.where(temp >= 0, temp, 0.01 * temp)


def _combine_final(acc, dent, asd, h, bias2):
    return pl.pallas_call(
        _combine_final_body,
        grid=(TC_GRID,),
        in_specs=[
            pl.BlockSpec((2, TC_BLK, CH), lambda i: (0, i, 0)),
            pl.BlockSpec((TC_BLK, NW), lambda i: (i, 0)),
            pl.BlockSpec((TC_BLK, 2), lambda i: (i, 0)),
            pl.BlockSpec((2, TC_BLK, CH), lambda i: (0, i, 0)),
            pl.BlockSpec((1, C), lambda i: (0, 0)),
        ],
        out_specs=pl.BlockSpec((TC_BLK, C), lambda i: (i, 0)),
        out_shape=jax.ShapeDtypeStruct((N, C), jnp.float32),
    )(acc, dent, asd, h, bias2)


_SC_MESH = plsc.VectorSubcoreMesh(core_axis_name="c", subcore_axis_name="s")


@functools.partial(
    pl.kernel,
    out_type=[
        jax.ShapeDtypeStruct((NC, N, CH), jnp.float32),
        jax.ShapeDtypeStruct((NW * N,), jnp.float32),
    ],
    mesh=_SC_MESH,
    compiler_params=pltpu.CompilerParams(needs_layout_passes=False,
                                         use_tc_tiling_on_sc=False),
    scratch_types=[
        pltpu.VMEM((B,), jnp.int32),
        pltpu.VMEM((B,), jnp.int32),
        pltpu.VMEM((B,), jnp.int32),
        pltpu.VMEM((B,), jnp.int32),
        pltpu.VMEM((B,), jnp.int32),
        pltpu.VMEM((B,), jnp.int32),
        pltpu.VMEM((2, TAIL), jnp.int32),
        pltpu.VMEM((N,), jnp.float32),
        pltpu.VMEM((N,), jnp.float32),
        pltpu.VMEM((N,), jnp.float32),
        pltpu.VMEM((B, CH), jnp.float32),
        pltpu.VMEM((B, CH), jnp.float32),
        pltpu.VMEM((B, CH), jnp.float32),
        pltpu.VMEM_SHARED((N, CH), jnp.float32),
        pltpu.VMEM_SHARED((N, CH), jnp.float32),
        pltpu.SemaphoreType.DMA,
        pltpu.SemaphoreType.DMA,
        pltpu.SemaphoreType.DMA,
        pltpu.SemaphoreType.DMA,
        pltpu.SemaphoreType.DMA,
        pltpu.SemaphoreType.DMA,
        pltpu.SemaphoreType.DMA,
        pltpu.SemaphoreType.DMA,
        pltpu.SemaphoreType.DMA,
        pltpu.SemaphoreType.DMA,
        pltpu.SemaphoreType.DMA,
        pltpu.SemaphoreType.DMA,
    ],
)
def _sc_edges(h_hbm, a_s_hbm, a_d_hbm, src_hbm, dst_hbm, ert_hbm,
              zacc_hbm, acc_out, den_out,
              srcv0, srcv1, srcv2, dstv0, dstv1, dstv2, idxt,
              as_v, ad_v, den_v, rows0, rows1, rows2,
              acc_sh, h_sh,
              gsem0, gsem1, gsem2, ssem0, ssem1, ssem2,
              isg0, isg1, isg2, isd0, isd1, isd2):
    c = lax.axis_index("c")
    s = lax.axis_index("s")
    wid = c * NS + s
    ebase = s * EPT

    pltpu.sync_copy(a_s_hbm, as_v)
    pltpu.sync_copy(a_d_hbm, ad_v)

    @pl.when(s < N // ZR)
    def _():
        pltpu.sync_copy(zacc_hbm.at[pl.ds(s * ZR, ZR)],
                        acc_sh.at[pl.ds(s * ZR, ZR)])
        pltpu.sync_copy(h_hbm.at[c, pl.ds(s * ZR, ZR)],
                        h_sh.at[pl.ds(s * ZR, ZR)])

    zero16 = jnp.zeros((16,), jnp.float32)

    @pl.loop(0, N // 16)
    def _(i):
        den_v[pl.ds(i * 16, 16)] = zero16

    plsc.subcore_barrier()

    rows = (rows0, rows1, rows2)
    srcv = (srcv0, srcv1, srcv2)
    dstv = (dstv0, dstv1, dstv2)
    gsem = (gsem0, gsem1, gsem2)
    ssem = (ssem0, ssem1, ssem2)
    isg = (isg0, isg1, isg2)
    isd = (isd0, isd1, isd2)

    def stage_src(b, i):
        pltpu.async_copy(src_hbm.at[pl.ds(ebase + b * B, B)], srcv[i], isg[i])

    def stage_dst(b, i):
        pltpu.async_copy(dst_hbm.at[pl.ds(ebase + b * B, B)], dstv[i], isd[i])

    def wait_src(i):
        pltpu.make_async_copy(src_hbm.at[pl.ds(0, B)], srcv[i], isg[i]).wait()

    def wait_dst(i):
        pltpu.make_async_copy(dst_hbm.at[pl.ds(0, B)], dstv[i], isd[i]).wait()

    def start_gather(i):
        pltpu.async_copy(h_sh.at[srcv[i]], rows[i], gsem[i])

    def wait_gather(i):
        pltpu.make_async_copy(h_sh.at[srcv[i]], rows[i], gsem[i]).wait()

    def start_scatter(i):
        pltpu.async_copy(rows[i], acc_sh.at[dstv[i]], ssem[i], add=True)

    def wait_scatter(i):
        pltpu.make_async_copy(rows[i], acc_sh.at[dstv[i]], ssem[i]).wait()

    def compute(src_b, dst_b, rows_v, ngroups=B // 16):
        for g in range(ngroups):
            off = g * 16
            s16 = src_b[pl.ds(off, 16)]
            d16 = dst_b[pl.ds(off, 16)]
            e = plsc.load_gather(as_v, [s16]) + plsc.load_gather(ad_v, [d16])
            e = jnp.where(e >= 0, e, 0.2 * e)
            ex = jnp.exp(e)
            plsc.addupdate_scatter(den_v, [d16], ex)
            for j in range(16):
                w = ex[j]
                r = off + j
                for lg in range(LG):
                    sl = pl.ds(lg * 16, 16)
                    rows_v[r, sl] = rows_v[r, sl] * w

    def head(i):
        wait_gather(i)
        wait_dst(i)
        compute(srcv[i], dstv[i], rows[i])
        start_scatter(i)

    def run(b, i):
        ip = (i + 2) % 3
        head(i)
        stage_src(b + 3, i)
        wait_scatter(ip)
        stage_dst(b + 2, ip)
        wait_src(ip)
        start_gather(ip)

    for i in range(3):
        stage_src(i, i)
        stage_dst(i, i)
    for i in range(3):
        wait_src(i)
        start_gather(i)

    head(0)
    stage_src(3, 0)

    @pl.loop(0, STEADY // 3)
    def _(t):
        b = 3 * t
        run(b + 1, 1)
        run(b + 2, 2)
        run(b + 3, 0)

    run(NBF - 5, 1)
    run(NBF - 4, 2)
    head(0)
    wait_scatter(2)
    stage_dst(NBF - 1, 2)
    wait_src(2)
    start_gather(2)
    head(1)
    wait_scatter(0)
    head(2)
    wait_scatter(1)
    wait_scatter(2)

    pltpu.sync_copy(ert_hbm.at[s], idxt)
    rowst = rows0.at[pl.ds(0, TAIL)]
    pltpu.async_copy(h_sh.at[idxt.at[0]], rowst, gsem0).wait()
    compute(idxt.at[0], idxt.at[1], rows0, ngroups=TAIL // 16)
    pltpu.async_copy(rowst, acc_sh.at[idxt.at[1]], ssem0, add=True).wait()

    pltpu.sync_copy(den_v, den_out.at[pl.ds(wid * N, N)])

    plsc.subcore_barrier()

    @pl.when(s < N // ZR)
    def _():
        pltpu.sync_copy(acc_sh.at[pl.ds(s * ZR, ZR)],
                        acc_out.at[c, pl.ds(s * ZR, ZR)])


def kernel(x, edges, W, att_src, att_dst, bias):
    att2 = jnp.stack([att_src, att_dst], axis=1)
    bias2 = bias.reshape(1, C)
    src_c = edges[0].reshape(NS, EPT)
    dst_c = edges[1].reshape(NS, EPT)
    ert = jnp.stack([src_c[:, NBF * B:], dst_c[:, NBF * B:]], axis=1)
    src_f = edges[0]
    dst_f = edges[1]
    zacc = jnp.zeros((N, CH), jnp.float32)

    h1h, asd1 = _proj(x, W, att2)
    acc1, den1 = _sc_edges(h1h, asd1[:, 0], asd1[:, 1], src_f, dst_f, ert,
                           zacc)
    h2h, asd2 = _combine_mid(acc1, den1.reshape(NW, N).T, asd1, h1h, bias2, W,
                             att2)
    acc2, den2 = _sc_edges(h2h, asd2[:, 0], asd2[:, 1], src_f, dst_f, ert,
                           zacc)
    return _combine_final(acc2, den2.reshape(NW, N).T, asd2, h2h, bias2)
